# baseline (speedup 1.0000x reference)
import jax
import jax.numpy as jnp
from jax.experimental import pallas as pl
from jax.experimental.pallas import tpu as pltpu

B = 16384
D = 128
C = 2048
NCH = B // C


def _body(x_hbm, gu_hbm, gi_hbm, xui_ref, xbuf, sem_in, sem_out):
    ins = []
    for k in range(NCH):
        cu = pltpu.make_async_copy(
            x_hbm.at[0, pl.ds(k * C, C), :], xbuf.at[0, pl.ds(k * C, C), :],
            sem_in.at[k])
        ci = pltpu.make_async_copy(
            x_hbm.at[1, pl.ds(k * C, C), :], xbuf.at[1, pl.ds(k * C, C), :],
            sem_in.at[k])
        cu.start()
        ci.start()
        ins.append((cu, ci))
    outs = []
    for k in range(NCH):
        for c in ins[k]:
            c.wait()
        cu = pltpu.make_async_copy(
            xbuf.at[0, pl.ds(k * C, C), :], gu_hbm.at[pl.ds(k * C, C), :],
            sem_out)
        ci = pltpu.make_async_copy(
            xbuf.at[1, pl.ds(k * C, C), :], gi_hbm.at[pl.ds(k * C, C), :],
            sem_out)
        cu.start()
        ci.start()
        outs.append(cu)
        outs.append(ci)
        gu = xbuf[0, pl.ds(k * C, C), :]
        gi = xbuf[1, pl.ds(k * C, C), :]
        xui_ref[pl.ds(k * (C // 128), C // 128), :] = (
            jnp.sum(gu * gi, axis=1).reshape(C // 128, 128))
    for c in outs:
        c.wait()


def kernel(inputs):
    gu_out, gi_out, xui2d = pl.pallas_call(
        _body,
        in_specs=[pl.BlockSpec(memory_space=pl.ANY)],
        out_specs=[
            pl.BlockSpec(memory_space=pl.ANY),
            pl.BlockSpec(memory_space=pl.ANY),
            pl.BlockSpec((B // 128, 128), lambda: (0, 0)),
        ],
        out_shape=[
            jax.ShapeDtypeStruct((B, D), jnp.float32),
            jax.ShapeDtypeStruct((B, D), jnp.float32),
            jax.ShapeDtypeStruct((B // 128, 128), jnp.float32),
        ],
        scratch_shapes=[
            pltpu.VMEM((2, B, D), jnp.float32),
            pltpu.SemaphoreType.DMA((NCH,)),
            pltpu.SemaphoreType.DMA,
        ],
    )(inputs)
    return (xui2d.reshape(B), gu_out, gi_out)


# speedup vs baseline: 1.0343x; 1.0343x over previous
import jax
import jax.numpy as jnp
from jax.experimental import pallas as pl
from jax.experimental.pallas import tpu as pltpu

B = 16384
D = 128
C = 4096
NCH = B // C


def _body(x_hbm, gu_hbm, gi_hbm, xui_ref, xbuf, sem_in, sem_out):
    ins = []
    for k in range(NCH):
        cu = pltpu.make_async_copy(
            x_hbm.at[0, pl.ds(k * C, C), :], xbuf.at[0, pl.ds(k * C, C), :],
            sem_in.at[k])
        ci = pltpu.make_async_copy(
            x_hbm.at[1, pl.ds(k * C, C), :], xbuf.at[1, pl.ds(k * C, C), :],
            sem_in.at[k])
        cu.start()
        ci.start()
        ins.append((cu, ci))
    outs = []
    for k in range(NCH):
        for c in ins[k]:
            c.wait()
        cu = pltpu.make_async_copy(
            xbuf.at[0, pl.ds(k * C, C), :], gu_hbm.at[pl.ds(k * C, C), :],
            sem_out)
        ci = pltpu.make_async_copy(
            xbuf.at[1, pl.ds(k * C, C), :], gi_hbm.at[pl.ds(k * C, C), :],
            sem_out)
        cu.start()
        ci.start()
        outs.append(cu)
        outs.append(ci)
        gu = xbuf[0, pl.ds(k * C, C), :]
        gi = xbuf[1, pl.ds(k * C, C), :]
        xui_ref[pl.ds(k * (C // 128), C // 128), :] = (
            jnp.sum(gu * gi, axis=1).reshape(C // 128, 128))
    for c in outs:
        c.wait()


def kernel(inputs):
    gu_out, gi_out, xui2d = pl.pallas_call(
        _body,
        in_specs=[pl.BlockSpec(memory_space=pl.ANY)],
        out_specs=[
            pl.BlockSpec(memory_space=pl.ANY),
            pl.BlockSpec(memory_space=pl.ANY),
            pl.BlockSpec((B // 128, 128), lambda: (0, 0)),
        ],
        out_shape=[
            jax.ShapeDtypeStruct((B, D), jnp.float32),
            jax.ShapeDtypeStruct((B, D), jnp.float32),
            jax.ShapeDtypeStruct((B // 128, 128), jnp.float32),
        ],
        scratch_shapes=[
            pltpu.VMEM((2, B, D), jnp.float32),
            pltpu.SemaphoreType.DMA((NCH,)),
            pltpu.SemaphoreType.DMA,
        ],
    )(inputs)
    return (xui2d.reshape(B), gu_out, gi_out)
